# baseline (speedup 1.0000x reference)
import jax
import jax.numpy as jnp
import numpy as np
from jax import lax
from jax.experimental import pallas as pl
from jax.experimental.pallas import tpu as pltpu
from jax.experimental.pallas import tpu_sc as plsc

D_MODEL = 128
VOCAB = 100000
B = 1024
L = 200

NC = 2
NS = 16
NW = NC * NS
ROWS = B * L
ROWS_PER_W = ROWS // NW
CHUNK = 200
NCHUNK = ROWS_PER_W // CHUNK
NBUF = 2
LANES = 16
DSLICES = D_MODEL // LANES


def _pe_table() -> np.ndarray:
    pos = np.arange(L, dtype=np.float32)[:, None]
    dim = np.arange(0, D_MODEL, 2, dtype=np.float32)
    angle = pos / np.power(10000.0, dim / D_MODEL)
    pe = np.zeros((L, D_MODEL), dtype=np.float32)
    pe[:, 0::2] = np.sin(angle)
    pe[:, 1::2] = np.cos(angle)
    return pe


_PE = _pe_table()


def _sc_body(x_hbm, pe_hbm, table_hbm, out_hbm, idx_v, pe_v, bufs, gsems, osems):
    wid = lax.axis_index("s") * NC + lax.axis_index("c")
    base = wid * ROWS_PER_W

    pltpu.sync_copy(x_hbm.at[pl.ds(base, ROWS_PER_W)], idx_v)
    pltpu.sync_copy(pe_hbm, pe_v)

    def start_gather(c, slot):
        return pltpu.async_copy(
            table_hbm.at[idx_v.at[pl.ds(c * CHUNK, CHUNK)]], bufs[slot], gsems[slot]
        )

    @pl.loop(0, NCHUNK, step=NBUF)
    def _group(c0):
        for b in range(NBUF):
            c = c0 + b
            cur = b

            @pl.when(c >= NBUF)
            def _():
                pltpu.make_async_copy(
                    bufs[cur],
                    out_hbm.at[pl.ds(base + (c - NBUF) * CHUNK, CHUNK)],
                    osems[cur],
                ).wait()

            start_gather(c, cur).wait()

            @pl.loop(0, CHUNK, unroll=2)
            def _row(r):
                for s in range(DSLICES):
                    sl = pl.ds(s * LANES, LANES)
                    bufs[cur][r, sl] += pe_v[r, sl]

            pltpu.async_copy(
                bufs[cur],
                out_hbm.at[pl.ds(base + c * CHUNK, CHUNK)],
                osems[cur],
            )

    for c in range(NCHUNK - NBUF, NCHUNK):
        pltpu.make_async_copy(
            bufs[c % NBUF],
            out_hbm.at[pl.ds(base + c * CHUNK, CHUNK)],
            osems[c % NBUF],
        ).wait()


@jax.jit
def _sc_embed(x_flat, pe, table):
    mesh = plsc.VectorSubcoreMesh(core_axis_name="c", subcore_axis_name="s")
    return pl.kernel(
        _sc_body,
        out_type=jax.ShapeDtypeStruct((ROWS, D_MODEL), jnp.float32),
        mesh=mesh,
        scratch_types=[
            pltpu.VMEM((ROWS_PER_W,), jnp.int32),
            pltpu.VMEM((L, D_MODEL), jnp.float32),
            [pltpu.VMEM((CHUNK, D_MODEL), jnp.float32) for _ in range(NBUF)],
            [pltpu.SemaphoreType.DMA for _ in range(NBUF)],
            [pltpu.SemaphoreType.DMA for _ in range(NBUF)],
        ],
    )(x_flat, pe, table)


def kernel(x, table):
    x_flat = x.reshape(ROWS).astype(jnp.int32)
    pe = jnp.asarray(_PE)
    out = _sc_embed(x_flat, pe, table)
    return out.reshape(B, L, D_MODEL)


# speedup vs baseline: 2.2015x; 2.2015x over previous
import jax
import jax.numpy as jnp
import numpy as np
from jax import lax
from jax.experimental import pallas as pl
from jax.experimental.pallas import tpu as pltpu
from jax.experimental.pallas import tpu_sc as plsc

D_MODEL = 128
VOCAB = 100000
B = 1024
L = 200

NC = 2
NS = 16
NW = NC * NS
ROWS = B * L
ROWS_PER_W = ROWS // NW
SEQ_PER_CHUNK = 2
CHUNK = SEQ_PER_CHUNK * L
NCHUNK = ROWS_PER_W // CHUNK
LANES = 16
DSLICES = D_MODEL // LANES


def _pe_table() -> np.ndarray:
    pos = np.arange(L, dtype=np.float32)[:, None]
    dim = np.arange(0, D_MODEL, 2, dtype=np.float32)
    angle = pos / np.power(10000.0, dim / D_MODEL)
    pe = np.zeros((L, D_MODEL), dtype=np.float32)
    pe[:, 0::2] = np.sin(angle)
    pe[:, 1::2] = np.cos(angle)
    return pe


_PE = _pe_table()


def _sc_body(x_hbm, pe_hbm, table_hbm, out_hbm, idx_v, pe_v, rows_v, sem):
    wid = lax.axis_index("s") * NC + lax.axis_index("c")
    base = wid * ROWS_PER_W

    pltpu.sync_copy(x_hbm.at[pl.ds(base, ROWS_PER_W)], idx_v)
    pltpu.sync_copy(pe_hbm, pe_v)

    @pl.loop(0, NCHUNK)
    def _chunk(c):
        off = c * CHUNK
        pltpu.async_copy(
            table_hbm.at[idx_v.at[pl.ds(off, CHUNK)]], rows_v, sem
        ).wait()

        @pl.loop(0, L, unroll=2)
        def _row(r):
            for s in range(DSLICES):
                sl = pl.ds(s * LANES, LANES)
                p = pe_v[r, sl]
                for q in range(SEQ_PER_CHUNK):
                    rows_v[q * L + r, sl] += p

        pltpu.sync_copy(rows_v, out_hbm.at[pl.ds(base + off, CHUNK)])


@jax.jit
def _sc_embed(x_flat, pe, table):
    mesh = plsc.VectorSubcoreMesh(core_axis_name="c", subcore_axis_name="s")
    return pl.kernel(
        _sc_body,
        out_type=jax.ShapeDtypeStruct((ROWS, D_MODEL), jnp.float32),
        mesh=mesh,
        scratch_types=[
            pltpu.VMEM((ROWS_PER_W,), jnp.int32),
            pltpu.VMEM((L, D_MODEL), jnp.float32),
            pltpu.VMEM((CHUNK, D_MODEL), jnp.float32),
            pltpu.SemaphoreType.DMA,
        ],
    )(x_flat, pe, table)


def kernel(x, table):
    x_flat = x.reshape(ROWS).astype(jnp.int32)
    pe = jnp.asarray(_PE)
    out = _sc_embed(x_flat, pe, table)
    return out.reshape(B, L, D_MODEL)


# speedup vs baseline: 2.2063x; 1.0022x over previous
import jax
import jax.numpy as jnp
import numpy as np
from jax import lax
from jax.experimental import pallas as pl
from jax.experimental.pallas import tpu as pltpu
from jax.experimental.pallas import tpu_sc as plsc

D_MODEL = 128
VOCAB = 100000
B = 1024
L = 200

NC = 2
NS = 16
NW = NC * NS
ROWS = B * L
ROWS_PER_W = ROWS // NW
SEQ_PER_CHUNK = 2
CHUNK = SEQ_PER_CHUNK * L
NCHUNK = ROWS_PER_W // CHUNK
LANES = 16
DSLICES = D_MODEL // LANES


def _pe_table() -> np.ndarray:
    pos = np.arange(L, dtype=np.float32)[:, None]
    dim = np.arange(0, D_MODEL, 2, dtype=np.float32)
    angle = pos / np.power(10000.0, dim / D_MODEL)
    pe = np.zeros((L, D_MODEL), dtype=np.float32)
    pe[:, 0::2] = np.sin(angle)
    pe[:, 1::2] = np.cos(angle)
    return pe


_PE = _pe_table()


def _sc_body(x_hbm, pe_hbm, table_hbm, out_hbm, idx_v, pe_v, rows_v, sem):
    wid = lax.axis_index("s") * NC + lax.axis_index("c")
    base = wid * ROWS_PER_W

    pltpu.sync_copy(x_hbm.at[pl.ds(base, ROWS_PER_W)], idx_v)
    pltpu.sync_copy(pe_hbm, pe_v)

    @pl.loop(0, NCHUNK)
    def _chunk(c):
        off = c * CHUNK
        pltpu.async_copy(
            table_hbm.at[idx_v.at[pl.ds(off, CHUNK)]], rows_v, sem
        ).wait()

        @pl.loop(0, L, unroll=2)
        def _row(r):
            for s in range(DSLICES):
                sl = pl.ds(s * LANES, LANES)
                p = pe_v[r, sl]
                for q in range(SEQ_PER_CHUNK):
                    plsc.addupdate(rows_v.at[q * L + r, sl], p)

        pltpu.sync_copy(rows_v, out_hbm.at[pl.ds(base + off, CHUNK)])


@jax.jit
def _sc_embed(x_flat, pe, table):
    mesh = plsc.VectorSubcoreMesh(core_axis_name="c", subcore_axis_name="s")
    return pl.kernel(
        _sc_body,
        out_type=jax.ShapeDtypeStruct((ROWS, D_MODEL), jnp.float32),
        mesh=mesh,
        scratch_types=[
            pltpu.VMEM((ROWS_PER_W,), jnp.int32),
            pltpu.VMEM((L, D_MODEL), jnp.float32),
            pltpu.VMEM((CHUNK, D_MODEL), jnp.float32),
            pltpu.SemaphoreType.DMA,
        ],
    )(x_flat, pe, table)


def kernel(x, table):
    x_flat = x.reshape(ROWS).astype(jnp.int32)
    pe = jnp.asarray(_PE)
    out = _sc_embed(x_flat, pe, table)
    return out.reshape(B, L, D_MODEL)
